# baseline (speedup 1.0000x reference)
import jax
import jax.numpy as jnp
from jax import lax
from jax.experimental import pallas as pl
from jax.experimental.pallas import tpu as pltpu
from jax.experimental.pallas import tpu_sc as plsc

HIST_LEN = 200
EMBED_DIM = 32
LANES = 128
OUT_LEN = (1 + HIST_LEN) * EMBED_DIM
HI_ROWS = 13
LO_ROWS = 12
END_ROWS = 11
MAX_ROWS = HI_ROWS


def _worker_base(w):
    if w < 9:
        return w * HI_ROWS
    if w < 15:
        return 9 * HI_ROWS + (w - 9) * LO_ROWS
    return 9 * HI_ROWS + 6 * LO_ROWS


def _extract_column(block, col, rowbuf, offset):
    for h in range(EMBED_DIM // 16):
        idx_d = lax.iota(jnp.int32, 16) + (h * 16)
        idx_c = jnp.zeros((16,), jnp.int32) + col
        vals = plsc.load_gather(block, [idx_d, idx_c])
        rowbuf[pl.ds(offset + h * 16, 16)] = vals


def _fetch_idx(movie_idx, idx_v, base, n_rows, sem):
    abase = (base // 8) * 8
    sh = base - abase
    cp = pltpu.async_copy(movie_idx.at[pl.ds(abase, sh + n_rows)],
                          idx_v.at[pl.ds(0, sh + n_rows)], sem)

    def getter():
        iv0 = idx_v[pl.ds(0, 16)]
        iv1 = idx_v[pl.ds(16, 16)] if sh + n_rows > 16 else None

        def get(k):
            return iv0[k] if k < 16 else iv1[k - 16]

        return get

    return cp, sh, getter


def _block_copies(get, sh, n_rows, tabT, blocks, sem):
    copies = []
    for j in range(n_rows):
        i = get(sh + j)
        t = pl.multiple_of((i // LANES) * LANES, LANES)
        copies.append(pltpu.async_copy(
            tabT.at[:, pl.ds(t, LANES)], blocks.at[j], sem))
    return copies


def _do_rows(n_rows, base, movie_idx, movie_tabT, out, idx_v, blocks,
             rowbuf, sem):
    cp, sh, getter = _fetch_idx(movie_idx, idx_v, base, n_rows, sem)
    cp.wait()
    get = getter()
    copies = _block_copies(get, sh, n_rows, movie_tabT, blocks, sem)
    for c in copies:
        c.wait()
    for j in range(n_rows):
        _extract_column(blocks.at[j], get(sh + j) % LANES, rowbuf,
                        j * EMBED_DIM)
    pltpu.sync_copy(
        rowbuf.at[pl.ds(0, n_rows * EMBED_DIM)],
        out.at[pl.ds(EMBED_DIM + base * EMBED_DIM, n_rows * EMBED_DIM)])


def _gather_body(user_idx, movie_idx, user_tabT, movie_tabT, out,
                 idx_v, uidx_v, blocks, ublock, rowbuf, urowbuf, sem, usem):
    wid = lax.axis_index("s")

    def _make_branch(w):
        n = HI_ROWS if w < 9 else LO_ROWS

        @pl.when(wid == w)
        def _():
            _do_rows(n, _worker_base(w), movie_idx, movie_tabT, out,
                     idx_v, blocks, rowbuf, sem)

    for w in range(15):
        _make_branch(w)

    @pl.when(wid == 15)
    def _():
        base = _worker_base(15)
        ucp = pltpu.async_copy(user_idx, uidx_v.at[pl.ds(0, 1)], usem)
        mcp, sh, getter = _fetch_idx(movie_idx, idx_v, base, END_ROWS, sem)
        ucp.wait()
        mcp.wait()
        get = getter()
        ui = uidx_v[...][0]
        ut = pl.multiple_of((ui // LANES) * LANES, LANES)
        ucopy = pltpu.async_copy(
            user_tabT.at[:, pl.ds(ut, LANES)], ublock, usem)
        copies = _block_copies(get, sh, END_ROWS, movie_tabT, blocks, sem)
        ucopy.wait()
        for c in copies:
            c.wait()
        _extract_column(ublock, ui % LANES, urowbuf, 0)
        pltpu.sync_copy(urowbuf, out.at[pl.ds(0, EMBED_DIM)])
        for j in range(END_ROWS):
            _extract_column(blocks.at[j], get(sh + j) % LANES, rowbuf,
                            j * EMBED_DIM)
        pltpu.sync_copy(
            rowbuf.at[pl.ds(0, END_ROWS * EMBED_DIM)],
            out.at[pl.ds(EMBED_DIM + base * EMBED_DIM,
                         END_ROWS * EMBED_DIM)])


@jax.jit
def kernel(user, movie_history, user_table, movie_table):
    mesh = plsc.VectorSubcoreMesh(
        core_axis_name="c", subcore_axis_name="s", num_cores=1)
    flat = pl.kernel(
        _gather_body,
        out_type=jax.ShapeDtypeStruct((OUT_LEN,), jnp.float32),
        mesh=mesh,
        scratch_types=[
            pltpu.VMEM((32,), jnp.int32),
            pltpu.VMEM((16,), jnp.int32),
            pltpu.VMEM((MAX_ROWS, EMBED_DIM, LANES), jnp.float32),
            pltpu.VMEM((EMBED_DIM, LANES), jnp.float32),
            pltpu.VMEM((MAX_ROWS * EMBED_DIM,), jnp.float32),
            pltpu.VMEM((EMBED_DIM,), jnp.float32),
            pltpu.SemaphoreType.DMA,
            pltpu.SemaphoreType.DMA,
        ],
        compiler_params=pltpu.CompilerParams(needs_layout_passes=False),
    )(user, movie_history, user_table.T, movie_table.T)
    return flat.reshape(1, OUT_LEN)
